# Initial kernel scaffold; baseline (speedup 1.0000x reference)
import functools

import jax
import jax.numpy as jnp
from jax import lax
from jax.experimental import pallas as pl
from jax.experimental.pallas import tpu as pltpu
from jax.experimental.pallas import tpu_sc as plsc

_N = 10000
_E = 320000
_D = 128
_DEPTH = 4

_NC = 2
_NS = 16
_NW = _NC * _NS
_CH = 128
_CHUNKS = 80
_EPT = _CH * _CHUNKS
_EPAD = _EPT * _NW
_NPAD = _N + 16
_ZROWS = _NPAD // _NS
_OROWS = _N // _NS

_mesh = plsc.VectorSubcoreMesh(
    core_axis_name="c", subcore_axis_name="s", num_cores=_NC, num_subcores=_NS
)


@functools.partial(
    pl.kernel,
    out_type=jax.ShapeDtypeStruct((_NC, _N, _D), jnp.float32),
    mesh=_mesh,
    scratch_types=[
        pltpu.VMEM((_CHUNKS, _CH), jnp.int32),
        pltpu.VMEM((_CHUNKS, _CH), jnp.int32),
        pltpu.VMEM((_CH, _D), jnp.float32),
        pltpu.VMEM((_CH, _D), jnp.float32),
        pltpu.VMEM_SHARED((_NPAD, _D), jnp.float32),
        pltpu.SemaphoreType.DMA,
        pltpu.SemaphoreType.DMA,
    ],
)
def _segsum_sc(src_hbm, dst_hbm, xx_hbm, zeros_hbm, out_hbm,
               src_v, dst_v, bufa, bufb, agg, sema, semb):
    c = lax.axis_index("c")
    s = lax.axis_index("s")
    wid = c * _NS + s
    pltpu.sync_copy(src_hbm.at[wid], src_v)
    pltpu.sync_copy(dst_hbm.at[wid], dst_v)
    pltpu.sync_copy(zeros_hbm, agg.at[pl.ds(s * _ZROWS, _ZROWS)])
    plsc.subcore_barrier()

    def body(jj, _):
        for b, (buf, sem) in enumerate(((bufa, sema), (bufb, semb))):
            j = jj * 2 + b
            pltpu.async_copy(xx_hbm.at[src_v.at[j]], buf, sem).wait()
            pltpu.sync_copy(buf, agg.at[dst_v.at[j]], add=True)
        return 0

    lax.fori_loop(0, _CHUNKS // 2, body, 0)
    plsc.subcore_barrier()
    pltpu.sync_copy(
        agg.at[pl.ds(s * _OROWS, _OROWS)],
        out_hbm.at[c].at[pl.ds(s * _OROWS, _OROWS)],
    )


def _gelu(x):
    return 0.5 * x * (1.0 + lax.erf(x * 0.7071067811865476))


def _foot_body(x_ref, w_ref, b_ref, o_ref):
    o_ref[...] = _gelu(
        jnp.dot(x_ref[...], w_ref[...], preferred_element_type=jnp.float32)
        + b_ref[...]
    )


def _mlp_body(xx_ref, pp_ref, w1_ref, b1_ref, w2_ref, b2_ref, o_ref):
    xx = xx_ref[...]
    u = xx + pp_ref[0] + pp_ref[1]
    u = _gelu(
        jnp.dot(u, w1_ref[...], preferred_element_type=jnp.float32) + b1_ref[...]
    )
    o_ref[...] = xx + (
        jnp.dot(u, w2_ref[...], preferred_element_type=jnp.float32) + b2_ref[...]
    )


def _head_body(xx_ref, w_ref, b_ref, o_ref):
    o_ref[...] = (
        jnp.dot(_gelu(xx_ref[...]), w_ref[...], preferred_element_type=jnp.float32)
        + b_ref[...]
    )


_R = 1000

_W_SPEC = pl.BlockSpec((_D, _D), lambda i: (0, 0))
_B_SPEC = pl.BlockSpec((1, _D), lambda i: (0, 0))
_X_SPEC = pl.BlockSpec((_R, _D), lambda i: (i, 0))

_foot_tc = pl.pallas_call(
    _foot_body,
    grid=(_N // _R,),
    in_specs=[_X_SPEC, _W_SPEC, _B_SPEC],
    out_specs=_X_SPEC,
    out_shape=jax.ShapeDtypeStruct((_N, _D), jnp.float32),
)

_mlp_tc = pl.pallas_call(
    _mlp_body,
    grid=(_N // _R,),
    in_specs=[
        _X_SPEC,
        pl.BlockSpec((_NC, _R, _D), lambda i: (0, i, 0)),
        _W_SPEC, _B_SPEC, _W_SPEC, _B_SPEC,
    ],
    out_specs=_X_SPEC,
    out_shape=jax.ShapeDtypeStruct((_N, _D), jnp.float32),
)

_head_tc = pl.pallas_call(
    _head_body,
    grid=(_N // _R,),
    in_specs=[_X_SPEC, _W_SPEC, _B_SPEC],
    out_specs=_X_SPEC,
    out_shape=jax.ShapeDtypeStruct((_N, _D), jnp.float32),
)


def kernel(x, edge_index, center, ptr, W_foot, b_foot, W1, b1, W2, b2, W_head, b_head):
    pad = _EPAD - _E
    src = jnp.concatenate([edge_index[0], jnp.zeros((pad,), jnp.int32)])
    dst = jnp.concatenate([edge_index[1], jnp.full((pad,), _N, jnp.int32)])
    src_r = src.reshape(_NW, _CHUNKS, _CH)
    dst_r = dst.reshape(_NW, _CHUNKS, _CH)
    zeros = jnp.zeros((_ZROWS, _D), jnp.float32)

    w_foot_t = W_foot.T
    w1_t = jnp.swapaxes(W1, 1, 2)
    w2_t = jnp.swapaxes(W2, 1, 2)
    w_head_t = jnp.zeros((_D, _D), jnp.float32).at[:, : W_head.shape[0]].set(W_head.T)
    b_head_p = jnp.zeros((1, _D), jnp.float32).at[0, : W_head.shape[0]].set(b_head)

    xx = _foot_tc(x, w_foot_t, b_foot.reshape(1, _D))
    for i in range(_DEPTH + 1):
        partials = _segsum_sc(src_r, dst_r, xx, zeros)
        xx = _mlp_tc(
            xx, partials,
            w1_t[i], b1[i].reshape(1, _D),
            w2_t[i], b2[i].reshape(1, _D),
        )
    out_full = _head_tc(xx, w_head_t, b_head_p)
    idx = center + ptr[:-1]
    return out_full[idx, :7]


# speedup vs baseline: 2.4717x; 2.4717x over previous
import functools

import jax
import jax.numpy as jnp
from jax import lax
from jax.experimental import pallas as pl
from jax.experimental.pallas import tpu as pltpu
from jax.experimental.pallas import tpu_sc as plsc

_N = 10000
_E = 320000
_D = 128
_DEPTH = 4

_NC = 2
_NS = 16
_CH = 128
_CHUNKS = 80
_EPT = _CH * _CHUNKS
_EPAD = _EPT * _NS * _NC
_NPAD = 10240
_ZROWS = _NPAD // _NS
_OROWS = 624

_mesh = plsc.VectorSubcoreMesh(
    core_axis_name="c", subcore_axis_name="s", num_cores=_NC, num_subcores=_NS
)


@functools.partial(
    pl.kernel,
    out_type=jax.ShapeDtypeStruct((_NC, _N, _D), jnp.float32),
    mesh=_mesh,
    scratch_types=[
        pltpu.VMEM((_CHUNKS, _CH), jnp.int32),
        pltpu.VMEM((_CHUNKS, _CH), jnp.int32),
        pltpu.VMEM((_CH, _D), jnp.float32),
        pltpu.VMEM_SHARED((_NPAD, _D), jnp.float32),
        pltpu.SemaphoreType.DMA,
    ],
)
def _segsum_sc(src_hbm, dst_hbm, xx_hbm, zeros_hbm, out_hbm,
               src_v, dst_v, buf, agg, sem):
    c = lax.axis_index("c")
    s = lax.axis_index("s")
    pltpu.sync_copy(src_hbm.at[c].at[s], src_v)
    pltpu.sync_copy(dst_hbm.at[c].at[s], dst_v)
    pltpu.sync_copy(zeros_hbm, agg.at[pl.ds(s * _ZROWS, _ZROWS)])
    plsc.subcore_barrier()

    def body(j, _):
        pltpu.async_copy(xx_hbm.at[src_v.at[j]], buf, sem).wait()
        pltpu.sync_copy(buf, agg.at[dst_v.at[j]], add=True)
        return 0

    lax.fori_loop(0, _CHUNKS, body, 0)
    plsc.subcore_barrier()

    @pl.when(s < _NS - 1)
    def _():
        pltpu.sync_copy(
            agg.at[pl.ds(s * _OROWS, _OROWS)],
            out_hbm.at[c].at[pl.ds(s * _OROWS, _OROWS)],
        )

    @pl.when(s == _NS - 1)
    def _():
        pltpu.sync_copy(
            agg.at[pl.ds((_NS - 1) * _OROWS, _N - (_NS - 1) * _OROWS)],
            out_hbm.at[c].at[pl.ds((_NS - 1) * _OROWS, _N - (_NS - 1) * _OROWS)],
        )


def _gelu(x):
    return 0.5 * x * (1.0 + lax.erf(x * 0.7071067811865476))


def _foot_body(x_ref, w_ref, b_ref, o_ref):
    o_ref[...] = _gelu(
        jnp.dot(x_ref[...], w_ref[...], preferred_element_type=jnp.float32)
        + b_ref[...]
    )


def _mlp_body(xx_ref, pp_ref, w1_ref, b1_ref, w2_ref, b2_ref, o_ref):
    xx = xx_ref[...]
    u = xx + pp_ref[0] + pp_ref[1]
    u = _gelu(
        jnp.dot(u, w1_ref[...], preferred_element_type=jnp.float32) + b1_ref[...]
    )
    o_ref[...] = (
        xx + jnp.dot(u, w2_ref[...], preferred_element_type=jnp.float32) + b2_ref[...]
    )


def _head_body(xx_ref, w_ref, b_ref, o_ref):
    o_ref[...] = (
        jnp.dot(_gelu(xx_ref[...]), w_ref[...], preferred_element_type=jnp.float32)
        + b_ref[...]
    )


_R = 1000

_W_SPEC = pl.BlockSpec((_D, _D), lambda i: (0, 0))
_B_SPEC = pl.BlockSpec((1, _D), lambda i: (0, 0))
_X_SPEC = pl.BlockSpec((_R, _D), lambda i: (i, 0))
_P_SPEC = pl.BlockSpec((_NC, _R, _D), lambda i: (0, i, 0))
_X_SHAPE = jax.ShapeDtypeStruct((_N, _D), jnp.float32)

_foot_tc = pl.pallas_call(
    _foot_body,
    grid=(_N // _R,),
    in_specs=[_X_SPEC, _W_SPEC, _B_SPEC],
    out_specs=_X_SPEC,
    out_shape=_X_SHAPE,
)

_mlp_tc = pl.pallas_call(
    _mlp_body,
    grid=(_N // _R,),
    in_specs=[_X_SPEC, _P_SPEC, _W_SPEC, _B_SPEC, _W_SPEC, _B_SPEC],
    out_specs=_X_SPEC,
    out_shape=_X_SHAPE,
)

_head_tc = pl.pallas_call(
    _head_body,
    grid=(_N // _R,),
    in_specs=[_X_SPEC, _W_SPEC, _B_SPEC],
    out_specs=_X_SPEC,
    out_shape=_X_SHAPE,
)


def kernel(x, edge_index, center, ptr, W_foot, b_foot, W1, b1, W2, b2, W_head, b_head):
    pad = _EPAD - _E
    src = jnp.concatenate([edge_index[0], jnp.zeros((pad,), jnp.int32)])
    dst = jnp.concatenate([edge_index[1], jnp.full((pad,), _N, jnp.int32)])
    src_r = src.reshape(_NC, _NS, _CHUNKS, _CH)
    dst_r = dst.reshape(_NC, _NS, _CHUNKS, _CH)
    zeros = jnp.zeros((_ZROWS, _D), jnp.float32)

    w_foot_t = W_foot.T
    w1_t = jnp.swapaxes(W1, 1, 2)
    w2_t = jnp.swapaxes(W2, 1, 2)
    w_head_t = jnp.zeros((_D, _D), jnp.float32).at[:, : W_head.shape[0]].set(W_head.T)
    b_head_p = jnp.zeros((1, _D), jnp.float32).at[0, : W_head.shape[0]].set(b_head)

    xx = _foot_tc(x, w_foot_t, b_foot.reshape(1, _D))
    for i in range(_DEPTH + 1):
        partials = _segsum_sc(src_r, dst_r, xx, zeros)
        xx = _mlp_tc(
            xx, partials,
            w1_t[i], b1[i].reshape(1, _D),
            w2_t[i], b2[i].reshape(1, _D),
        )
    out_full = _head_tc(xx, w_head_t, b_head_p)
    idx = center + ptr[:-1]
    return out_full[idx, :7]


# speedup vs baseline: 2.7007x; 1.0926x over previous
import functools

import jax
import jax.numpy as jnp
from jax import lax
from jax.experimental import pallas as pl
from jax.experimental.pallas import tpu as pltpu
from jax.experimental.pallas import tpu_sc as plsc

_N = 10000
_E = 320000
_D = 128
_DEPTH = 4

_NC = 2
_NS = 16
_CH = 128
_CHUNKS = 80
_HALF = _CHUNKS // 2
_EPT = _CH * _CHUNKS
_EPAD = _EPT * _NS * _NC
_NPAD = 10240
_ZROWS = _NPAD // _NS
_OROWS = 624

_mesh = plsc.VectorSubcoreMesh(
    core_axis_name="c", subcore_axis_name="s", num_cores=_NC, num_subcores=_NS
)


@functools.partial(
    pl.kernel,
    out_type=jax.ShapeDtypeStruct((_NC, _N, _D), jnp.float32),
    mesh=_mesh,
    scratch_types=[
        pltpu.VMEM((_HALF, _CH), jnp.int32),
        pltpu.VMEM((_HALF, _CH), jnp.int32),
        pltpu.VMEM((_CH, _D), jnp.float32),
        pltpu.VMEM((_CH, _D), jnp.float32),
        pltpu.VMEM_SHARED((_NPAD, _D), jnp.float32),
        pltpu.SemaphoreType.DMA,
        pltpu.SemaphoreType.DMA,
    ],
)
def _segsum_sc(src_hbm, dst_hbm, xx_hbm, zeros_hbm, out_hbm,
               src_v, dst_v, bufa, bufb, agg, sema, semb):
    c = lax.axis_index("c")
    s = lax.axis_index("s")
    pltpu.sync_copy(zeros_hbm, agg.at[pl.ds(s * _ZROWS, _ZROWS)])
    plsc.subcore_barrier()

    def _start(row, buf, sem):
        pltpu.async_copy(xx_hbm.at[src_v.at[row]], buf, sem)

    def _wait(buf, sem):
        pltpu.make_async_copy(xx_hbm.at[src_v.at[0]], buf, sem).wait()

    for p in range(_CHUNKS // _HALF):
        pltpu.sync_copy(src_hbm.at[c].at[s].at[pl.ds(p * _HALF, _HALF)], src_v)
        pltpu.sync_copy(dst_hbm.at[c].at[s].at[pl.ds(p * _HALF, _HALF)], dst_v)
        _start(0, bufa, sema)

        def body(jj, _):
            _start(2 * jj + 1, bufb, semb)
            _wait(bufa, sema)
            pltpu.sync_copy(bufa, agg.at[dst_v.at[2 * jj]], add=True)

            @pl.when(jj < _HALF // 2 - 1)
            def _():
                _start(2 * jj + 2, bufa, sema)

            _wait(bufb, semb)
            pltpu.sync_copy(bufb, agg.at[dst_v.at[2 * jj + 1]], add=True)
            return 0

        lax.fori_loop(0, _HALF // 2, body, 0)
    plsc.subcore_barrier()

    @pl.when(s < _NS - 1)
    def _():
        pltpu.sync_copy(
            agg.at[pl.ds(s * _OROWS, _OROWS)],
            out_hbm.at[c].at[pl.ds(s * _OROWS, _OROWS)],
        )

    @pl.when(s == _NS - 1)
    def _():
        pltpu.sync_copy(
            agg.at[pl.ds((_NS - 1) * _OROWS, _N - (_NS - 1) * _OROWS)],
            out_hbm.at[c].at[pl.ds((_NS - 1) * _OROWS, _N - (_NS - 1) * _OROWS)],
        )


def _gelu(x):
    return 0.5 * x * (1.0 + lax.erf(x * 0.7071067811865476))


def _foot_body(x_ref, w_ref, b_ref, o_ref):
    o_ref[...] = _gelu(
        jnp.dot(x_ref[...], w_ref[...], preferred_element_type=jnp.float32)
        + b_ref[...]
    )


def _mlp_body(xx_ref, pp_ref, w1_ref, b1_ref, w2_ref, b2_ref, o_ref):
    xx = xx_ref[...]
    u = xx + pp_ref[0] + pp_ref[1]
    u = _gelu(
        jnp.dot(u, w1_ref[...], preferred_element_type=jnp.float32) + b1_ref[...]
    )
    o_ref[...] = (
        xx + jnp.dot(u, w2_ref[...], preferred_element_type=jnp.float32) + b2_ref[...]
    )


def _head_body(xx_ref, w_ref, b_ref, o_ref):
    o_ref[...] = (
        jnp.dot(_gelu(xx_ref[...]), w_ref[...], preferred_element_type=jnp.float32)
        + b_ref[...]
    )


_R = 1000

_W_SPEC = pl.BlockSpec((_D, _D), lambda i: (0, 0))
_B_SPEC = pl.BlockSpec((1, _D), lambda i: (0, 0))
_X_SPEC = pl.BlockSpec((_R, _D), lambda i: (i, 0))
_P_SPEC = pl.BlockSpec((_NC, _R, _D), lambda i: (0, i, 0))
_X_SHAPE = jax.ShapeDtypeStruct((_N, _D), jnp.float32)

_foot_tc = pl.pallas_call(
    _foot_body,
    grid=(_N // _R,),
    in_specs=[_X_SPEC, _W_SPEC, _B_SPEC],
    out_specs=_X_SPEC,
    out_shape=_X_SHAPE,
)

_mlp_tc = pl.pallas_call(
    _mlp_body,
    grid=(_N // _R,),
    in_specs=[_X_SPEC, _P_SPEC, _W_SPEC, _B_SPEC, _W_SPEC, _B_SPEC],
    out_specs=_X_SPEC,
    out_shape=_X_SHAPE,
)

_head_tc = pl.pallas_call(
    _head_body,
    grid=(_N // _R,),
    in_specs=[_X_SPEC, _W_SPEC, _B_SPEC],
    out_specs=_X_SPEC,
    out_shape=_X_SHAPE,
)


def kernel(x, edge_index, center, ptr, W_foot, b_foot, W1, b1, W2, b2, W_head, b_head):
    pad = _EPAD - _E
    src = jnp.concatenate([edge_index[0], jnp.zeros((pad,), jnp.int32)])
    dst = jnp.concatenate([edge_index[1], jnp.full((pad,), _N, jnp.int32)])
    src_r = src.reshape(_NC, _NS, _CHUNKS, _CH)
    dst_r = dst.reshape(_NC, _NS, _CHUNKS, _CH)
    zeros = jnp.zeros((_ZROWS, _D), jnp.float32)

    w_foot_t = W_foot.T
    w1_t = jnp.swapaxes(W1, 1, 2)
    w2_t = jnp.swapaxes(W2, 1, 2)
    w_head_t = jnp.zeros((_D, _D), jnp.float32).at[:, : W_head.shape[0]].set(W_head.T)
    b_head_p = jnp.zeros((1, _D), jnp.float32).at[0, : W_head.shape[0]].set(b_head)

    xx = _foot_tc(x, w_foot_t, b_foot.reshape(1, _D))
    for i in range(_DEPTH + 1):
        partials = _segsum_sc(src_r, dst_r, xx, zeros)
        xx = _mlp_tc(
            xx, partials,
            w1_t[i], b1[i].reshape(1, _D),
            w2_t[i], b2[i].reshape(1, _D),
        )
    out_full = _head_tc(xx, w_head_t, b_head_p)
    idx = center + ptr[:-1]
    return out_full[idx, :7]
